# baseline (speedup 1.0000x reference)
import functools

import jax
import jax.numpy as jnp
from jax import lax
from jax.experimental import pallas as pl
from jax.experimental.pallas import tpu as pltpu
from jax.experimental.pallas import tpu_sc as plsc

_E = 64
_K = 8
_COEF = 0.02
_N = 8 * 8192
_NC, _NS, _L = 2, 16, 16
_NW = _NC * _NS
_TOK_PER_W = _N // _NW
_CHUNK_TOK = 256
_CHUNK_WORDS = _CHUNK_TOK * _E
_N_CHUNKS = _TOK_PER_W // _CHUNK_TOK
_GROUPS = _CHUNK_TOK // _L

_SORT8 = (
    (0, 1), (2, 3), (4, 5), (6, 7),
    (0, 2), (1, 3), (4, 6), (5, 7),
    (1, 2), (5, 6),
    (0, 4), (1, 5), (2, 6), (3, 7),
    (2, 4), (3, 5),
    (1, 2), (3, 4), (5, 6),
)
_BITONIC8_DESC = (
    (0, 4), (1, 5), (2, 6), (3, 7),
    (0, 2), (1, 3), (4, 6), (5, 7),
    (0, 1), (2, 3), (4, 5), (6, 7),
)


def _sort8_asc(vs):
    vs = list(vs)
    for i, j in _SORT8:
        lo = jnp.minimum(vs[i], vs[j])
        vs[j] = jnp.maximum(vs[i], vs[j])
        vs[i] = lo
    return vs


def _merge_top8(run_desc, blk_asc):
    t = [jnp.maximum(run_desc[i], blk_asc[i]) for i in range(_K)]
    for i, j in _BITONIC8_DESC:
        hi = jnp.maximum(t[i], t[j])
        t[j] = jnp.minimum(t[i], t[j])
        t[i] = hi
    return t


def _sc_loss_kernel(x_hbm, out_hbm, buf0, buf1, accv, sem0, sem1):
    wid = lax.axis_index("s") * _NC + lax.axis_index("c")
    w_tok = wid * _TOK_PER_W

    one = jnp.ones((_L,), jnp.int32)
    lane_iota = lax.iota(jnp.int32, _L)
    row_step = jnp.full((_L,), _L, jnp.int32)
    mask63 = jnp.full((_L,), _E - 1, jnp.int32)
    scale = jnp.float32(_COEF * _E / (_N * _K))
    bufs = (buf0, buf1)
    sems = (sem0, sem1)

    def make_load8(buf):
        def load8(row, off):
            vs = []
            for _ in range(_K):
                vs.append(plsc.load_gather(buf, [row, off]))
                off = (off + one) & mask63
            return vs, off
        return load8

    def make_group_loop(buf):
        load8 = make_load8(buf)

        def run_groups(acc0):
            def group_body(g, carry):
                row, acc = carry
                blk, off = load8(row, lane_iota)
                asc = _sort8_asc(blk)
                run = asc[::-1]
                for _ in range(_E // _K - 1):
                    blk, off = load8(row, off)
                    run = _merge_top8(run, _sort8_asc(blk))
                m = run[0]
                s8 = jnp.exp(run[0] - m)
                for i in range(1, _K):
                    s8 = s8 + jnp.exp(run[i] - m)
                return row + row_step, acc + s8 / s8

            _, acc = lax.fori_loop(0, _GROUPS, group_body, (lane_iota, acc0))
            return acc
        return run_groups

    group_loops = (make_group_loop(buf0), make_group_loop(buf1))

    def start_chunk(c, which):
        return pltpu.async_copy(
            x_hbm.at[pl.ds(w_tok + c * _CHUNK_TOK, _CHUNK_TOK), :],
            bufs[which], sems[which])

    def wait_chunk(which):
        pltpu.make_async_copy(x_hbm.at[pl.ds(0, _CHUNK_TOK), :],
                              bufs[which], sems[which]).wait()

    last_c = jnp.int32(_N_CHUNKS - 1)
    start_chunk(0, 0)
    start_chunk(1, 1)

    def chunk_pair_body(i, acc):
        wait_chunk(0)
        acc = group_loops[0](acc)
        start_chunk(jnp.minimum(2 * i + 2, last_c), 0)
        wait_chunk(1)
        acc = group_loops[1](acc)
        start_chunk(jnp.minimum(2 * i + 3, last_c), 1)
        return acc

    acc = lax.fori_loop(0, _N_CHUNKS // 2, chunk_pair_body,
                        jnp.zeros((_L,), jnp.float32))
    wait_chunk(0)
    wait_chunk(1)
    accv[...] = acc * scale
    pltpu.sync_copy(accv, out_hbm.at[pl.ds(wid * _L, _L)])


def kernel(gate_logits):
    logits = gate_logits.reshape(-1, _E)
    mesh = plsc.VectorSubcoreMesh(core_axis_name="c", subcore_axis_name="s")
    partials = pl.kernel(
        _sc_loss_kernel,
        mesh=mesh,
        compiler_params=pltpu.CompilerParams(needs_layout_passes=False),
        out_type=jax.ShapeDtypeStruct((_NW * _L,), jnp.float32),
        scratch_types=[
            pltpu.VMEM((_CHUNK_TOK, _E), jnp.float32),
            pltpu.VMEM((_CHUNK_TOK, _E), jnp.float32),
            pltpu.VMEM((_L,), jnp.float32),
            pltpu.SemaphoreType.DMA,
            pltpu.SemaphoreType.DMA,
        ],
    )(logits)
    return jnp.sum(partials)


# speedup vs baseline: 1.1231x; 1.1231x over previous
import functools

import jax
import jax.numpy as jnp
from jax import lax
from jax.experimental import pallas as pl
from jax.experimental.pallas import tpu as pltpu
from jax.experimental.pallas import tpu_sc as plsc

_E = 64
_K = 8
_COEF = 0.02
_N = 8 * 8192
_NC, _NS, _L = 2, 16, 16
_NW = _NC * _NS
_TOK_PER_W = _N // _NW
_CHUNK_TOK = 256
_CHUNK_WORDS = _CHUNK_TOK * _E
_N_CHUNKS = _TOK_PER_W // _CHUNK_TOK
_GROUPS = _CHUNK_TOK // _L

_SORT8 = (
    (0, 1), (2, 3), (4, 5), (6, 7),
    (0, 2), (1, 3), (4, 6), (5, 7),
    (1, 2), (5, 6),
    (0, 4), (1, 5), (2, 6), (3, 7),
    (2, 4), (3, 5),
    (1, 2), (3, 4), (5, 6),
)
_BITONIC8_DESC = (
    (0, 4), (1, 5), (2, 6), (3, 7),
    (0, 2), (1, 3), (4, 6), (5, 7),
    (0, 1), (2, 3), (4, 5), (6, 7),
)


def _sort8_asc(vs):
    vs = list(vs)
    for i, j in _SORT8:
        lo = jnp.minimum(vs[i], vs[j])
        vs[j] = jnp.maximum(vs[i], vs[j])
        vs[i] = lo
    return vs


def _merge_top8(run_desc, blk_asc):
    t = [jnp.maximum(run_desc[i], blk_asc[i]) for i in range(_K)]
    for i, j in _BITONIC8_DESC:
        hi = jnp.maximum(t[i], t[j])
        t[j] = jnp.minimum(t[i], t[j])
        t[i] = hi
    return t


def _sc_loss_kernel(x_hbm, out_hbm, buf0, buf1, accv, sem0, sem1):
    wid = lax.axis_index("s") * _NC + lax.axis_index("c")
    w_tok = wid * _TOK_PER_W

    one = jnp.ones((_L,), jnp.int32)
    lane_iota = lax.iota(jnp.int32, _L)
    row_step = jnp.full((_L,), _L, jnp.int32)
    mask63 = jnp.full((_L,), _E - 1, jnp.int32)
    scale = jnp.float32(_COEF * _E / (_N * _K))
    bufs = (buf0, buf1)
    sems = (sem0, sem1)

    def make_load8(buf):
        def load8(row, off):
            vs = []
            for _ in range(_K):
                vs.append(plsc.load_gather(buf, [row, off]))
                off = (off + one) & mask63
            return vs, off
        return load8

    def make_group_loop(buf):
        load8 = make_load8(buf)

        def run_groups(acc0):
            def group_body(g, carry):
                row, acc = carry
                blk, off = load8(row, lane_iota)
                asc = _sort8_asc(blk)
                run = asc[::-1]
                for _ in range(_E // _K - 1):
                    blk, off = load8(row, off)
                    run = _merge_top8(run, _sort8_asc(blk))
                m = run[0]
                s8 = jnp.exp(run[0] - m)
                for i in range(1, _K):
                    s8 = s8 + jnp.exp(run[i] - m)
                return row + row_step, acc + s8 / s8

            def ploop_body(g, carry):
                return group_body(g, carry)

            _, acc = plsc.parallel_loop(0, _GROUPS, carry=(lane_iota, acc0),
                                        unroll=2)(ploop_body)
            return acc
        return run_groups

    group_loops = (make_group_loop(buf0), make_group_loop(buf1))

    def start_chunk(c):
        return pltpu.async_copy(
            x_hbm.at[pl.ds(w_tok + c * _CHUNK_TOK, _CHUNK_TOK), :],
            bufs[c % 2], sems[c % 2])

    copies = [start_chunk(0)]
    acc = jnp.zeros((_L,), jnp.float32)
    for c in range(_N_CHUNKS):
        copies[c].wait()
        if c + 1 < _N_CHUNKS:
            copies.append(start_chunk(c + 1))
        acc = group_loops[c % 2](acc)
    accv[...] = acc * scale
    pltpu.sync_copy(accv, out_hbm.at[pl.ds(wid * _L, _L)])


def kernel(gate_logits):
    logits = gate_logits.reshape(-1, _E)
    mesh = plsc.VectorSubcoreMesh(core_axis_name="c", subcore_axis_name="s")
    partials = pl.kernel(
        _sc_loss_kernel,
        mesh=mesh,
        compiler_params=pltpu.CompilerParams(needs_layout_passes=False),
        out_type=jax.ShapeDtypeStruct((_NW * _L,), jnp.float32),
        scratch_types=[
            pltpu.VMEM((_CHUNK_TOK, _E), jnp.float32),
            pltpu.VMEM((_CHUNK_TOK, _E), jnp.float32),
            pltpu.VMEM((_L,), jnp.float32),
            pltpu.SemaphoreType.DMA,
            pltpu.SemaphoreType.DMA,
        ],
    )(logits)
    return jnp.sum(partials)
